# baseline (speedup 1.0000x reference)
import functools

import jax
import jax.numpy as jnp
from jax import lax
from jax.experimental import pallas as pl
from jax.experimental.pallas import tpu as pltpu
from jax.experimental.pallas import tpu_sc as plsc

_BUCKET_NUM = 50
_FEATURE_DIM = 64
_N = 262144
_CLIP_MIN, _CLIP_MAX = 0.1, 10.0
_START_SMOOTH = 1

_NC, _NS = 2, 16
_NW = _NC * _NS
_SPW = _N // _NW
_C = 128
_NCHUNK = _SPW // _C
_L = 16
_TBL = _BUCKET_NUM * _FEATURE_DIM


def _nr_sqrt(r):
    i = lax.bitcast_convert_type(r, jnp.int32)
    i = jnp.int32(0x5F3759DF) - lax.shift_right_arithmetic(i, jnp.int32(1))
    y = lax.bitcast_convert_type(i, jnp.float32)
    for _ in range(3):
        y = y * (1.5 - 0.5 * r * y * y)
    return r * y


def _sc_body(feat_hbm, lab_hbm, edges_hbm, gate_hbm, m1_hbm, v1_hbm, m2_hbm, v2_hbm,
             out_hbm,
             m1_v, s_v, m2_v, b_v, edges_v, gate_v, lab_v,
             feat_v0, feat_v1, out_v0, out_v1, si0, si1, so0, so1):
    wid = lax.axis_index("s") * _NC + lax.axis_index("c")
    base = wid * _SPW

    def in_slice(c):
        return feat_hbm.at[pl.ds(base + c * _C, _C)]

    def out_slice(c):
        return out_hbm.at[pl.ds((base + c * _C) * _FEATURE_DIM, _C * _FEATURE_DIM)]

    pltpu.async_copy(in_slice(0), feat_v0, si0)

    pltpu.sync_copy(edges_hbm, edges_v)
    pltpu.sync_copy(gate_hbm, gate_v)
    pltpu.sync_copy(m1_hbm, m1_v)
    pltpu.sync_copy(v1_hbm, s_v)
    pltpu.sync_copy(m2_hbm, m2_v)
    pltpu.sync_copy(v2_hbm, b_v)
    pltpu.sync_copy(lab_hbm.at[pl.ds(base, _SPW)], lab_v)

    gv = gate_v[pl.ds(0, _L)]

    def tbl_body(i, _):
        sl = pl.ds(i * _L, _L)
        v1 = s_v[sl]
        v2 = b_v[sl]
        m1 = m1_v[sl]
        m2 = m2_v[sl]
        r = jnp.minimum(jnp.maximum(v2 / v1, _CLIP_MIN), _CLIP_MAX)
        s_raw = _nr_sqrt(r)
        s_v[sl] = 1.0 + gv * (s_raw - 1.0)
        b_v[sl] = gv * (m2 - m1 * s_raw)
        return 0

    lax.fori_loop(0, _TBL // _L, tbl_body, 0)

    def compute_chunk(c, feat_v, out_v):
        def group_body(gi, _):
            s0 = c * _C + gi * _L
            labv = lab_v[pl.ds(s0, _L)]
            k0 = labv * 10.0
            k0 = jnp.minimum(jnp.maximum(k0.astype(jnp.int32), 0), _BUCKET_NUM - 1)
            e_lo = plsc.load_gather(edges_v, [k0])
            e_hi = plsc.load_gather(edges_v, [k0 + 1])
            b16 = k0 - (labv < e_lo).astype(jnp.int32) + (labv >= e_hi).astype(jnp.int32)
            b16 = jnp.minimum(jnp.maximum(b16, 0), _BUCKET_NUM - 1)
            rowb16 = b16 * _FEATURE_DIM
            l0 = gi * _L
            for i in range(_L):
                rb = rowb16[i]
                r = l0 + i
                ob = r * _FEATURE_DIM
                for j in range(_FEATURE_DIM // _L):
                    x = feat_v[r, pl.ds(j * _L, _L)]
                    s = s_v[pl.ds(rb + j * _L, _L)]
                    b = b_v[pl.ds(rb + j * _L, _L)]
                    out_v[pl.ds(ob + j * _L, _L)] = x * s + b
            return 0

        lax.fori_loop(0, _C // _L, group_body, 0)

    def pair_body(g, _):
        c0 = g * 2
        pltpu.async_copy(in_slice(c0 + 1), feat_v1, si1)
        pltpu.make_async_copy(in_slice(c0), feat_v0, si0).wait()

        @pl.when(g > 0)
        def _():
            pltpu.make_async_copy(out_v0, out_slice(c0 - 2), so0).wait()

        compute_chunk(c0, feat_v0, out_v0)
        pltpu.async_copy(out_v0, out_slice(c0), so0)

        @pl.when(c0 + 2 < _NCHUNK)
        def _():
            pltpu.async_copy(in_slice(c0 + 2), feat_v0, si0)

        pltpu.make_async_copy(in_slice(c0 + 1), feat_v1, si1).wait()

        @pl.when(g > 0)
        def _():
            pltpu.make_async_copy(out_v1, out_slice(c0 - 1), so1).wait()

        compute_chunk(c0 + 1, feat_v1, out_v1)
        pltpu.async_copy(out_v1, out_slice(c0 + 1), so1)
        return 0

    lax.fori_loop(0, _NCHUNK // 2, pair_body, 0)
    pltpu.make_async_copy(out_v0, out_slice(_NCHUNK - 2), so0).wait()
    pltpu.make_async_copy(out_v1, out_slice(_NCHUNK - 1), so1).wait()


@jax.jit
def _fds_smooth(features, labels, gate, m1, v1, m2, v2):
    edges = jnp.linspace(0.0, 5.0, _BUCKET_NUM + 1)
    edges_pad = jnp.concatenate([edges, jnp.full((13,), 3.0e38, jnp.float32)])
    mesh = plsc.VectorSubcoreMesh(core_axis_name="c", subcore_axis_name="s")
    run = pl.kernel(
        _sc_body,
        out_type=jax.ShapeDtypeStruct((_N * _FEATURE_DIM,), jnp.float32),
        mesh=mesh,
        compiler_params=pltpu.CompilerParams(needs_layout_passes=False,
                                             use_tc_tiling_on_sc=True),
        scratch_types=[
            pltpu.VMEM((_TBL,), jnp.float32),
            pltpu.VMEM((_TBL,), jnp.float32),
            pltpu.VMEM((_TBL,), jnp.float32),
            pltpu.VMEM((_TBL,), jnp.float32),
            pltpu.VMEM((64,), jnp.float32),
            pltpu.VMEM((_L,), jnp.float32),
            pltpu.VMEM((_SPW,), jnp.float32),
            pltpu.VMEM((_C, _FEATURE_DIM), jnp.float32),
            pltpu.VMEM((_C, _FEATURE_DIM), jnp.float32),
            pltpu.VMEM((_C * _FEATURE_DIM,), jnp.float32),
            pltpu.VMEM((_C * _FEATURE_DIM,), jnp.float32),
            pltpu.SemaphoreType.DMA,
            pltpu.SemaphoreType.DMA,
            pltpu.SemaphoreType.DMA,
            pltpu.SemaphoreType.DMA,
        ],
    )
    out_flat = run(
        features,
        labels.reshape(_N),
        edges_pad,
        jnp.full((_L,), gate, jnp.float32),
        m1.reshape(_TBL),
        v1.reshape(_TBL),
        m2.reshape(_TBL),
        v2.reshape(_TBL),
    )
    return out_flat.reshape(_N, _FEATURE_DIM)


def kernel(features, labels, epoch, running_mean_last_epoch, running_var_last_epoch,
           smoothed_mean_last_epoch, smoothed_var_last_epoch):
    gate = jnp.where(epoch < _START_SMOOTH, 0.0, 1.0).astype(jnp.float32)
    return _fds_smooth(features, labels, gate,
                       running_mean_last_epoch, running_var_last_epoch,
                       smoothed_mean_last_epoch, smoothed_var_last_epoch)


# speedup vs baseline: 1.1989x; 1.1989x over previous
import jax
import jax.numpy as jnp
from jax import lax
from jax.experimental import pallas as pl
from jax.experimental.pallas import tpu as pltpu
from jax.experimental.pallas import tpu_sc as plsc

_BUCKET_NUM = 50
_FEATURE_DIM = 64
_N = 262144
_CLIP_MIN, _CLIP_MAX = 0.1, 10.0
_START_SMOOTH = 1

_NC, _NS = 2, 16
_NW = _NC * _NS
_SPW = _N // _NW
_L = 16
_TBL = _BUCKET_NUM * _FEATURE_DIM
_TPAD = 64
_B = 1024


def _nr_sqrt(r):
    i = lax.bitcast_convert_type(r, jnp.int32)
    i = jnp.int32(0x5F3759DF) - lax.shift_right_arithmetic(i, jnp.int32(1))
    y = lax.bitcast_convert_type(i, jnp.float32)
    for _ in range(3):
        y = y * (1.5 - 0.5 * r * y * y)
    return r * y


def _sc_body(lab_hbm, edges_hbm, gate_hbm, m1_hbm, v1_hbm, m2_hbm, v2_hbm,
             idx_hbm, scale_hbm, bias_hbm,
             m1_v, s_v, m2_v, b_v, edges_v, gate_v, lab_v, idx_v, s2_v, b2_v):
    wid = lax.axis_index("s") * _NC + lax.axis_index("c")
    base = wid * _SPW

    pltpu.sync_copy(edges_hbm, edges_v)
    pltpu.sync_copy(lab_hbm.at[pl.ds(base, _SPW)], lab_v)

    @pl.when(wid == 0)
    def _():
        pltpu.sync_copy(gate_hbm, gate_v)
        pltpu.sync_copy(m1_hbm, m1_v)
        pltpu.sync_copy(v1_hbm, s_v)
        pltpu.sync_copy(m2_hbm, m2_v)
        pltpu.sync_copy(v2_hbm, b_v)
        gv = gate_v[pl.ds(0, _L)]

        def tbl_body(i, _):
            sl = pl.ds(i * _L, _L)
            v1 = s_v[sl]
            v2 = b_v[sl]
            m1 = m1_v[sl]
            m2 = m2_v[sl]
            r = jnp.minimum(jnp.maximum(v2 / v1, _CLIP_MIN), _CLIP_MAX)
            s_raw = _nr_sqrt(r)
            row = i // (_FEATURE_DIM // _L)
            col = (i % (_FEATURE_DIM // _L)) * _L
            s2_v[row, pl.ds(col, _L)] = 1.0 + gv * (s_raw - 1.0)
            b2_v[row, pl.ds(col, _L)] = gv * (m2 - m1 * s_raw)
            return 0

        lax.fori_loop(0, _TBL // _L, tbl_body, 0)

        z = jnp.zeros((_L,), jnp.float32)

        def pad_body(i, _):
            row = _BUCKET_NUM + i // (_FEATURE_DIM // _L)
            col = (i % (_FEATURE_DIM // _L)) * _L
            s2_v[row, pl.ds(col, _L)] = z
            b2_v[row, pl.ds(col, _L)] = z
            return 0

        lax.fori_loop(0, (_TPAD - _BUCKET_NUM) * (_FEATURE_DIM // _L), pad_body, 0)
        pltpu.sync_copy(s2_v, scale_hbm)
        pltpu.sync_copy(b2_v, bias_hbm)

    def group_body(gi, _):
        s0 = gi * _L
        labv = lab_v[pl.ds(s0, _L)]
        k0 = labv * 10.0
        k0 = jnp.minimum(jnp.maximum(k0.astype(jnp.int32), 0), _BUCKET_NUM - 1)
        e_lo = plsc.load_gather(edges_v, [k0])
        e_hi = plsc.load_gather(edges_v, [k0 + 1])
        b16 = k0 - (labv < e_lo).astype(jnp.int32) + (labv >= e_hi).astype(jnp.int32)
        b16 = jnp.minimum(jnp.maximum(b16, 0), _BUCKET_NUM - 1)
        idx_v[pl.ds(s0, _L)] = b16
        return 0

    lax.fori_loop(0, _SPW // _L, group_body, 0)
    pltpu.sync_copy(idx_v, idx_hbm.at[pl.ds(base, _SPW)])


def _tc_body(f_ref, i_ref, s_ref, b_ref, o_ref):
    idxb = i_ref[...]
    cols = lax.broadcasted_iota(jnp.int32, (_B, _TPAD), 1)
    oh = (idxb[:, None] == cols).astype(jnp.float32)
    rs = jnp.dot(oh, s_ref[...], preferred_element_type=jnp.float32)
    rb = jnp.dot(oh, b_ref[...], preferred_element_type=jnp.float32)
    o_ref[...] = f_ref[...] * rs + rb


@jax.jit
def _fds_smooth(features, labels, gate, m1, v1, m2, v2):
    edges = jnp.linspace(0.0, 5.0, _BUCKET_NUM + 1)
    edges_pad = jnp.concatenate([edges, jnp.full((13,), 3.0e38, jnp.float32)])
    mesh = plsc.VectorSubcoreMesh(core_axis_name="c", subcore_axis_name="s")
    bin_kernel = pl.kernel(
        _sc_body,
        out_type=(
            jax.ShapeDtypeStruct((_N,), jnp.int32),
            jax.ShapeDtypeStruct((_TPAD, _FEATURE_DIM), jnp.float32),
            jax.ShapeDtypeStruct((_TPAD, _FEATURE_DIM), jnp.float32),
        ),
        mesh=mesh,
        compiler_params=pltpu.CompilerParams(needs_layout_passes=False,
                                             use_tc_tiling_on_sc=True),
        scratch_types=[
            pltpu.VMEM((_TBL,), jnp.float32),
            pltpu.VMEM((_TBL,), jnp.float32),
            pltpu.VMEM((_TBL,), jnp.float32),
            pltpu.VMEM((_TBL,), jnp.float32),
            pltpu.VMEM((64,), jnp.float32),
            pltpu.VMEM((_L,), jnp.float32),
            pltpu.VMEM((_SPW,), jnp.float32),
            pltpu.VMEM((_SPW,), jnp.int32),
            pltpu.VMEM((_TPAD, _FEATURE_DIM), jnp.float32),
            pltpu.VMEM((_TPAD, _FEATURE_DIM), jnp.float32),
        ],
    )
    idx, scale_t, bias_t = bin_kernel(
        labels.reshape(_N),
        edges_pad,
        jnp.full((_L,), gate, jnp.float32),
        m1.reshape(_TBL),
        v1.reshape(_TBL),
        m2.reshape(_TBL),
        v2.reshape(_TBL),
    )
    return pl.pallas_call(
        _tc_body,
        grid=(_N // _B,),
        in_specs=[
            pl.BlockSpec((_B, _FEATURE_DIM), lambda i: (i, 0)),
            pl.BlockSpec((_B,), lambda i: (i,)),
            pl.BlockSpec((_TPAD, _FEATURE_DIM), lambda i: (0, 0)),
            pl.BlockSpec((_TPAD, _FEATURE_DIM), lambda i: (0, 0)),
        ],
        out_specs=pl.BlockSpec((_B, _FEATURE_DIM), lambda i: (i, 0)),
        out_shape=jax.ShapeDtypeStruct((_N, _FEATURE_DIM), jnp.float32),
        compiler_params=pltpu.CompilerParams(
            dimension_semantics=("arbitrary",)),
    )(features, idx, scale_t, bias_t)


def kernel(features, labels, epoch, running_mean_last_epoch, running_var_last_epoch,
           smoothed_mean_last_epoch, smoothed_var_last_epoch):
    gate = jnp.where(epoch < _START_SMOOTH, 0.0, 1.0).astype(jnp.float32)
    return _fds_smooth(features, labels, gate,
                       running_mean_last_epoch, running_var_last_epoch,
                       smoothed_mean_last_epoch, smoothed_var_last_epoch)


# speedup vs baseline: 1.6279x; 1.3578x over previous
import jax
import jax.numpy as jnp
from jax import lax
from jax.experimental import pallas as pl
from jax.experimental.pallas import tpu as pltpu
from jax.experimental.pallas import tpu_sc as plsc

_BUCKET_NUM = 50
_FEATURE_DIM = 64
_N = 262144
_CLIP_MIN, _CLIP_MAX = 0.1, 10.0
_START_SMOOTH = 1

_NC, _NS = 2, 16
_NW = _NC * _NS
_SPW = _N // _NW
_L = 16
_TBL = _BUCKET_NUM * _FEATURE_DIM
_TPAD = 64
_B = 4096


def _nr_sqrt(r):
    i = lax.bitcast_convert_type(r, jnp.int32)
    i = jnp.int32(0x5F3759DF) - lax.shift_right_arithmetic(i, jnp.int32(1))
    y = lax.bitcast_convert_type(i, jnp.float32)
    for _ in range(3):
        y = y * (1.5 - 0.5 * r * y * y)
    return r * y


def _sc_body(lab_hbm, edges_hbm, gate_hbm, m1_hbm, v1_hbm, m2_hbm, v2_hbm,
             idx_hbm, scale_hbm, bias_hbm,
             m1_v, s_v, m2_v, b_v, edges_v, gate_v, lab_v, idx_v, s2_v, b2_v):
    wid = lax.axis_index("s") * _NC + lax.axis_index("c")
    base = wid * _SPW

    pltpu.sync_copy(edges_hbm, edges_v)
    pltpu.sync_copy(lab_hbm.at[pl.ds(base, _SPW)], lab_v)

    @pl.when(wid == 0)
    def _():
        pltpu.sync_copy(gate_hbm, gate_v)
        pltpu.sync_copy(m1_hbm, m1_v)
        pltpu.sync_copy(v1_hbm, s_v)
        pltpu.sync_copy(m2_hbm, m2_v)
        pltpu.sync_copy(v2_hbm, b_v)
        gv = gate_v[pl.ds(0, _L)]

        def tbl_body(i, _):
            sl = pl.ds(i * _L, _L)
            v1 = s_v[sl]
            v2 = b_v[sl]
            m1 = m1_v[sl]
            m2 = m2_v[sl]
            r = jnp.minimum(jnp.maximum(v2 / v1, _CLIP_MIN), _CLIP_MAX)
            s_raw = _nr_sqrt(r)
            row = i // (_FEATURE_DIM // _L)
            col = (i % (_FEATURE_DIM // _L)) * _L
            s2_v[row, pl.ds(col, _L)] = 1.0 + gv * (s_raw - 1.0)
            b2_v[row, pl.ds(col, _L)] = gv * (m2 - m1 * s_raw)
            return 0

        lax.fori_loop(0, _TBL // _L, tbl_body, 0)

        z = jnp.zeros((_L,), jnp.float32)

        def pad_body(i, _):
            row = _BUCKET_NUM + i // (_FEATURE_DIM // _L)
            col = (i % (_FEATURE_DIM // _L)) * _L
            s2_v[row, pl.ds(col, _L)] = z
            b2_v[row, pl.ds(col, _L)] = z
            return 0

        lax.fori_loop(0, (_TPAD - _BUCKET_NUM) * (_FEATURE_DIM // _L), pad_body, 0)
        pltpu.sync_copy(s2_v, scale_hbm)
        pltpu.sync_copy(b2_v, bias_hbm)

    def group_body(gi, _):
        s0 = gi * _L
        labv = lab_v[pl.ds(s0, _L)]
        k0 = labv * 10.0
        k0 = jnp.minimum(jnp.maximum(k0.astype(jnp.int32), 0), _BUCKET_NUM - 1)
        e_lo = plsc.load_gather(edges_v, [k0])
        e_hi = plsc.load_gather(edges_v, [k0 + 1])
        b16 = k0 - (labv < e_lo).astype(jnp.int32) + (labv >= e_hi).astype(jnp.int32)
        b16 = jnp.minimum(jnp.maximum(b16, 0), _BUCKET_NUM - 1)
        idx_v[pl.ds(s0, _L)] = b16
        return 0

    lax.fori_loop(0, _SPW // _L, group_body, 0)
    pltpu.sync_copy(idx_v, idx_hbm.at[pl.ds(base, _SPW)])


def _tc_body(f_ref, i_ref, s_ref, b_ref, o_ref):
    idxb = i_ref[...]
    cols = lax.broadcasted_iota(jnp.int32, (_B, _TPAD), 1)
    oh = (idxb[:, None] == cols).astype(jnp.float32)
    rs = jnp.dot(oh, s_ref[...], preferred_element_type=jnp.float32)
    rb = jnp.dot(oh, b_ref[...], preferred_element_type=jnp.float32)
    o_ref[...] = f_ref[...] * rs + rb


@jax.jit
def _fds_smooth(features, labels, gate, m1, v1, m2, v2):
    edges = jnp.linspace(0.0, 5.0, _BUCKET_NUM + 1)
    edges_pad = jnp.concatenate([edges, jnp.full((13,), 3.0e38, jnp.float32)])
    mesh = plsc.VectorSubcoreMesh(core_axis_name="c", subcore_axis_name="s")
    bin_kernel = pl.kernel(
        _sc_body,
        out_type=(
            jax.ShapeDtypeStruct((_N,), jnp.int32),
            jax.ShapeDtypeStruct((_TPAD, _FEATURE_DIM), jnp.float32),
            jax.ShapeDtypeStruct((_TPAD, _FEATURE_DIM), jnp.float32),
        ),
        mesh=mesh,
        compiler_params=pltpu.CompilerParams(needs_layout_passes=False,
                                             use_tc_tiling_on_sc=True),
        scratch_types=[
            pltpu.VMEM((_TBL,), jnp.float32),
            pltpu.VMEM((_TBL,), jnp.float32),
            pltpu.VMEM((_TBL,), jnp.float32),
            pltpu.VMEM((_TBL,), jnp.float32),
            pltpu.VMEM((64,), jnp.float32),
            pltpu.VMEM((_L,), jnp.float32),
            pltpu.VMEM((_SPW,), jnp.float32),
            pltpu.VMEM((_SPW,), jnp.int32),
            pltpu.VMEM((_TPAD, _FEATURE_DIM), jnp.float32),
            pltpu.VMEM((_TPAD, _FEATURE_DIM), jnp.float32),
        ],
    )
    idx, scale_t, bias_t = bin_kernel(
        labels.reshape(_N),
        edges_pad,
        jnp.full((_L,), gate, jnp.float32),
        m1.reshape(_TBL),
        v1.reshape(_TBL),
        m2.reshape(_TBL),
        v2.reshape(_TBL),
    )
    return pl.pallas_call(
        _tc_body,
        grid=(_N // _B,),
        in_specs=[
            pl.BlockSpec((_B, _FEATURE_DIM), lambda i: (i, 0)),
            pl.BlockSpec((_B,), lambda i: (i,)),
            pl.BlockSpec((_TPAD, _FEATURE_DIM), lambda i: (0, 0)),
            pl.BlockSpec((_TPAD, _FEATURE_DIM), lambda i: (0, 0)),
        ],
        out_specs=pl.BlockSpec((_B, _FEATURE_DIM), lambda i: (i, 0)),
        out_shape=jax.ShapeDtypeStruct((_N, _FEATURE_DIM), jnp.float32),
        compiler_params=pltpu.CompilerParams(
            dimension_semantics=("arbitrary",)),
    )(features, idx, scale_t, bias_t)


def kernel(features, labels, epoch, running_mean_last_epoch, running_var_last_epoch,
           smoothed_mean_last_epoch, smoothed_var_last_epoch):
    gate = jnp.where(epoch < _START_SMOOTH, 0.0, 1.0).astype(jnp.float32)
    return _fds_smooth(features, labels, gate,
                       running_mean_last_epoch, running_var_last_epoch,
                       smoothed_mean_last_epoch, smoothed_var_last_epoch)


# speedup vs baseline: 1.7355x; 1.0661x over previous
import jax
import jax.numpy as jnp
from jax import lax
from jax.experimental import pallas as pl
from jax.experimental.pallas import tpu as pltpu
from jax.experimental.pallas import tpu_sc as plsc

_BUCKET_NUM = 50
_FEATURE_DIM = 64
_N = 262144
_CLIP_MIN, _CLIP_MAX = 0.1, 10.0
_START_SMOOTH = 1

_NC, _NS = 2, 16
_NW = _NC * _NS
_SPW = _N // _NW
_L = 16
_TBL = _BUCKET_NUM * _FEATURE_DIM
_TPAD = 64
_B = 8192


def _nr_sqrt(r):
    i = lax.bitcast_convert_type(r, jnp.int32)
    i = jnp.int32(0x5F3759DF) - lax.shift_right_arithmetic(i, jnp.int32(1))
    y = lax.bitcast_convert_type(i, jnp.float32)
    for _ in range(3):
        y = y * (1.5 - 0.5 * r * y * y)
    return r * y


def _sc_body(lab_hbm, edges_hbm, gate_hbm, m1_hbm, v1_hbm, m2_hbm, v2_hbm,
             idx_hbm, scale_hbm, bias_hbm,
             m1_v, s_v, m2_v, b_v, edges_v, gate_v, lab_v, idx_v, s2_v, b2_v):
    wid = lax.axis_index("s") * _NC + lax.axis_index("c")
    base = wid * _SPW

    pltpu.sync_copy(edges_hbm, edges_v)
    pltpu.sync_copy(lab_hbm.at[pl.ds(base, _SPW)], lab_v)

    @pl.when(wid == 0)
    def _():
        pltpu.sync_copy(gate_hbm, gate_v)
        pltpu.sync_copy(m1_hbm, m1_v)
        pltpu.sync_copy(v1_hbm, s_v)
        pltpu.sync_copy(m2_hbm, m2_v)
        pltpu.sync_copy(v2_hbm, b_v)
        gv = gate_v[pl.ds(0, _L)]

        def tbl_body(i, _):
            sl = pl.ds(i * _L, _L)
            v1 = s_v[sl]
            v2 = b_v[sl]
            m1 = m1_v[sl]
            m2 = m2_v[sl]
            r = jnp.minimum(jnp.maximum(v2 / v1, _CLIP_MIN), _CLIP_MAX)
            s_raw = _nr_sqrt(r)
            row = i // (_FEATURE_DIM // _L)
            col = (i % (_FEATURE_DIM // _L)) * _L
            s2_v[row, pl.ds(col, _L)] = 1.0 + gv * (s_raw - 1.0)
            b2_v[row, pl.ds(col, _L)] = gv * (m2 - m1 * s_raw)
            return 0

        lax.fori_loop(0, _TBL // _L, tbl_body, 0)

        z = jnp.zeros((_L,), jnp.float32)

        def pad_body(i, _):
            row = _BUCKET_NUM + i // (_FEATURE_DIM // _L)
            col = (i % (_FEATURE_DIM // _L)) * _L
            s2_v[row, pl.ds(col, _L)] = z
            b2_v[row, pl.ds(col, _L)] = z
            return 0

        lax.fori_loop(0, (_TPAD - _BUCKET_NUM) * (_FEATURE_DIM // _L), pad_body, 0)
        pltpu.sync_copy(s2_v, scale_hbm)
        pltpu.sync_copy(b2_v, bias_hbm)

    def group_body(gi, _):
        s0 = gi * _L
        labv = lab_v[pl.ds(s0, _L)]
        k0 = labv * 10.0
        k0 = jnp.minimum(jnp.maximum(k0.astype(jnp.int32), 0), _BUCKET_NUM - 1)
        e_lo = plsc.load_gather(edges_v, [k0])
        e_hi = plsc.load_gather(edges_v, [k0 + 1])
        b16 = k0 - (labv < e_lo).astype(jnp.int32) + (labv >= e_hi).astype(jnp.int32)
        b16 = jnp.minimum(jnp.maximum(b16, 0), _BUCKET_NUM - 1)
        idx_v[pl.ds(s0, _L)] = b16
        return 0

    lax.fori_loop(0, _SPW // _L, group_body, 0)
    pltpu.sync_copy(idx_v, idx_hbm.at[pl.ds(base, _SPW)])


def _tc_body(f_ref, i_ref, s_ref, b_ref, o_ref):
    idxb = i_ref[...]
    cols = lax.broadcasted_iota(jnp.int32, (_B, _TPAD), 1)
    oh = (idxb[:, None] == cols).astype(jnp.float32)
    rs = jnp.dot(oh, s_ref[...], preferred_element_type=jnp.float32)
    rb = jnp.dot(oh, b_ref[...], preferred_element_type=jnp.float32)
    o_ref[...] = f_ref[...] * rs + rb


@jax.jit
def _fds_smooth(features, labels, gate, m1, v1, m2, v2):
    edges = jnp.linspace(0.0, 5.0, _BUCKET_NUM + 1)
    edges_pad = jnp.concatenate([edges, jnp.full((13,), 3.0e38, jnp.float32)])
    mesh = plsc.VectorSubcoreMesh(core_axis_name="c", subcore_axis_name="s")
    bin_kernel = pl.kernel(
        _sc_body,
        out_type=(
            jax.ShapeDtypeStruct((_N,), jnp.int32),
            jax.ShapeDtypeStruct((_TPAD, _FEATURE_DIM), jnp.float32),
            jax.ShapeDtypeStruct((_TPAD, _FEATURE_DIM), jnp.float32),
        ),
        mesh=mesh,
        compiler_params=pltpu.CompilerParams(needs_layout_passes=False,
                                             use_tc_tiling_on_sc=True),
        scratch_types=[
            pltpu.VMEM((_TBL,), jnp.float32),
            pltpu.VMEM((_TBL,), jnp.float32),
            pltpu.VMEM((_TBL,), jnp.float32),
            pltpu.VMEM((_TBL,), jnp.float32),
            pltpu.VMEM((64,), jnp.float32),
            pltpu.VMEM((_L,), jnp.float32),
            pltpu.VMEM((_SPW,), jnp.float32),
            pltpu.VMEM((_SPW,), jnp.int32),
            pltpu.VMEM((_TPAD, _FEATURE_DIM), jnp.float32),
            pltpu.VMEM((_TPAD, _FEATURE_DIM), jnp.float32),
        ],
    )
    idx, scale_t, bias_t = bin_kernel(
        labels.reshape(_N),
        edges_pad,
        jnp.full((_L,), gate, jnp.float32),
        m1.reshape(_TBL),
        v1.reshape(_TBL),
        m2.reshape(_TBL),
        v2.reshape(_TBL),
    )
    return pl.pallas_call(
        _tc_body,
        grid=(_N // _B,),
        in_specs=[
            pl.BlockSpec((_B, _FEATURE_DIM), lambda i: (i, 0)),
            pl.BlockSpec((_B,), lambda i: (i,)),
            pl.BlockSpec((_TPAD, _FEATURE_DIM), lambda i: (0, 0)),
            pl.BlockSpec((_TPAD, _FEATURE_DIM), lambda i: (0, 0)),
        ],
        out_specs=pl.BlockSpec((_B, _FEATURE_DIM), lambda i: (i, 0)),
        out_shape=jax.ShapeDtypeStruct((_N, _FEATURE_DIM), jnp.float32),
        compiler_params=pltpu.CompilerParams(
            dimension_semantics=("arbitrary",),
            vmem_limit_bytes=100 * 1024 * 1024),
    )(features, idx, scale_t, bias_t)


def kernel(features, labels, epoch, running_mean_last_epoch, running_var_last_epoch,
           smoothed_mean_last_epoch, smoothed_var_last_epoch):
    gate = jnp.where(epoch < _START_SMOOTH, 0.0, 1.0).astype(jnp.float32)
    return _fds_smooth(features, labels, gate,
                       running_mean_last_epoch, running_var_last_epoch,
                       smoothed_mean_last_epoch, smoothed_var_last_epoch)


# speedup vs baseline: 1.7423x; 1.0040x over previous
import jax
import jax.numpy as jnp
from jax import lax
from jax.experimental import pallas as pl
from jax.experimental.pallas import tpu as pltpu
from jax.experimental.pallas import tpu_sc as plsc

_BUCKET_NUM = 50
_FEATURE_DIM = 64
_N = 262144
_CLIP_MIN, _CLIP_MAX = 0.1, 10.0
_START_SMOOTH = 1

_NC, _NS = 2, 16
_NW = _NC * _NS
_SPW = _N // _NW
_L = 16
_TBL = _BUCKET_NUM * _FEATURE_DIM
_TPAD = 64
_B = 16384


def _nr_sqrt(r):
    i = lax.bitcast_convert_type(r, jnp.int32)
    i = jnp.int32(0x5F3759DF) - lax.shift_right_arithmetic(i, jnp.int32(1))
    y = lax.bitcast_convert_type(i, jnp.float32)
    for _ in range(3):
        y = y * (1.5 - 0.5 * r * y * y)
    return r * y


def _sc_body(lab_hbm, edges_hbm, gate_hbm, m1_hbm, v1_hbm, m2_hbm, v2_hbm,
             idx_hbm, scale_hbm, bias_hbm,
             m1_v, s_v, m2_v, b_v, edges_v, gate_v, lab_v, idx_v, s2_v, b2_v):
    wid = lax.axis_index("s") * _NC + lax.axis_index("c")
    base = wid * _SPW

    pltpu.sync_copy(edges_hbm, edges_v)
    pltpu.sync_copy(lab_hbm.at[pl.ds(base, _SPW)], lab_v)

    @pl.when(wid == 0)
    def _():
        pltpu.sync_copy(gate_hbm, gate_v)
        pltpu.sync_copy(m1_hbm, m1_v)
        pltpu.sync_copy(v1_hbm, s_v)
        pltpu.sync_copy(m2_hbm, m2_v)
        pltpu.sync_copy(v2_hbm, b_v)
        gv = gate_v[pl.ds(0, _L)]

        def tbl_body(i, _):
            sl = pl.ds(i * _L, _L)
            v1 = s_v[sl]
            v2 = b_v[sl]
            m1 = m1_v[sl]
            m2 = m2_v[sl]
            r = jnp.minimum(jnp.maximum(v2 / v1, _CLIP_MIN), _CLIP_MAX)
            s_raw = _nr_sqrt(r)
            row = i // (_FEATURE_DIM // _L)
            col = (i % (_FEATURE_DIM // _L)) * _L
            s2_v[row, pl.ds(col, _L)] = 1.0 + gv * (s_raw - 1.0)
            b2_v[row, pl.ds(col, _L)] = gv * (m2 - m1 * s_raw)
            return 0

        lax.fori_loop(0, _TBL // _L, tbl_body, 0)

        z = jnp.zeros((_L,), jnp.float32)

        def pad_body(i, _):
            row = _BUCKET_NUM + i // (_FEATURE_DIM // _L)
            col = (i % (_FEATURE_DIM // _L)) * _L
            s2_v[row, pl.ds(col, _L)] = z
            b2_v[row, pl.ds(col, _L)] = z
            return 0

        lax.fori_loop(0, (_TPAD - _BUCKET_NUM) * (_FEATURE_DIM // _L), pad_body, 0)
        pltpu.sync_copy(s2_v, scale_hbm)
        pltpu.sync_copy(b2_v, bias_hbm)

    def group_body(gi, _):
        s0 = gi * _L
        labv = lab_v[pl.ds(s0, _L)]
        k0 = labv * 10.0
        k0 = jnp.minimum(jnp.maximum(k0.astype(jnp.int32), 0), _BUCKET_NUM - 1)
        e_lo = plsc.load_gather(edges_v, [k0])
        e_hi = plsc.load_gather(edges_v, [k0 + 1])
        b16 = k0 - (labv < e_lo).astype(jnp.int32) + (labv >= e_hi).astype(jnp.int32)
        b16 = jnp.minimum(jnp.maximum(b16, 0), _BUCKET_NUM - 1)
        idx_v[pl.ds(s0, _L)] = b16
        return 0

    lax.fori_loop(0, _SPW // _L, group_body, 0)
    pltpu.sync_copy(idx_v, idx_hbm.at[pl.ds(base, _SPW)])


def _tc_body(f_ref, i_ref, s_ref, b_ref, o_ref):
    idxb = i_ref[...]
    cols = lax.broadcasted_iota(jnp.int32, (_B, _TPAD), 1)
    oh = (idxb[:, None] == cols).astype(jnp.float32)
    rs = jnp.dot(oh, s_ref[...], preferred_element_type=jnp.float32)
    rb = jnp.dot(oh, b_ref[...], preferred_element_type=jnp.float32)
    o_ref[...] = f_ref[...] * rs + rb


@jax.jit
def _fds_smooth(features, labels, gate, m1, v1, m2, v2):
    edges = jnp.linspace(0.0, 5.0, _BUCKET_NUM + 1)
    edges_pad = jnp.concatenate([edges, jnp.full((13,), 3.0e38, jnp.float32)])
    mesh = plsc.VectorSubcoreMesh(core_axis_name="c", subcore_axis_name="s")
    bin_kernel = pl.kernel(
        _sc_body,
        out_type=(
            jax.ShapeDtypeStruct((_N,), jnp.int32),
            jax.ShapeDtypeStruct((_TPAD, _FEATURE_DIM), jnp.float32),
            jax.ShapeDtypeStruct((_TPAD, _FEATURE_DIM), jnp.float32),
        ),
        mesh=mesh,
        compiler_params=pltpu.CompilerParams(needs_layout_passes=False,
                                             use_tc_tiling_on_sc=True),
        scratch_types=[
            pltpu.VMEM((_TBL,), jnp.float32),
            pltpu.VMEM((_TBL,), jnp.float32),
            pltpu.VMEM((_TBL,), jnp.float32),
            pltpu.VMEM((_TBL,), jnp.float32),
            pltpu.VMEM((64,), jnp.float32),
            pltpu.VMEM((_L,), jnp.float32),
            pltpu.VMEM((_SPW,), jnp.float32),
            pltpu.VMEM((_SPW,), jnp.int32),
            pltpu.VMEM((_TPAD, _FEATURE_DIM), jnp.float32),
            pltpu.VMEM((_TPAD, _FEATURE_DIM), jnp.float32),
        ],
    )
    idx, scale_t, bias_t = bin_kernel(
        labels.reshape(_N),
        edges_pad,
        jnp.full((_L,), gate, jnp.float32),
        m1.reshape(_TBL),
        v1.reshape(_TBL),
        m2.reshape(_TBL),
        v2.reshape(_TBL),
    )
    return pl.pallas_call(
        _tc_body,
        grid=(_N // _B,),
        in_specs=[
            pl.BlockSpec((_B, _FEATURE_DIM), lambda i: (i, 0)),
            pl.BlockSpec((_B,), lambda i: (i,)),
            pl.BlockSpec((_TPAD, _FEATURE_DIM), lambda i: (0, 0)),
            pl.BlockSpec((_TPAD, _FEATURE_DIM), lambda i: (0, 0)),
        ],
        out_specs=pl.BlockSpec((_B, _FEATURE_DIM), lambda i: (i, 0)),
        out_shape=jax.ShapeDtypeStruct((_N, _FEATURE_DIM), jnp.float32),
        compiler_params=pltpu.CompilerParams(
            dimension_semantics=("arbitrary",),
            vmem_limit_bytes=100 * 1024 * 1024),
    )(features, idx, scale_t, bias_t)


def kernel(features, labels, epoch, running_mean_last_epoch, running_var_last_epoch,
           smoothed_mean_last_epoch, smoothed_var_last_epoch):
    gate = jnp.where(epoch < _START_SMOOTH, 0.0, 1.0).astype(jnp.float32)
    return _fds_smooth(features, labels, gate,
                       running_mean_last_epoch, running_var_last_epoch,
                       smoothed_mean_last_epoch, smoothed_var_last_epoch)
